# baseline (speedup 1.0000x reference)
import functools
import math

import jax
import jax.numpy as jnp
from jax import lax
from jax.experimental import pallas as pl
from jax.experimental.pallas import tpu as pltpu
from jax.experimental.pallas import tpu_sc as plsc

D_MODEL = 64
SCALE = math.sqrt(D_MODEL)

_NC = 2
_NS = 16
_L = 16
_NW = _NC * _NS

_NBUF = 4
_LEAD = 2
_GMAX = 128


@functools.lru_cache(maxsize=None)
def _make_lookup(vocab, n_rows, n_cols):
    rows_per_w = n_rows // _NW
    splits = []
    c0 = 0
    while c0 < n_cols:
        g = min(_GMAX, n_cols - c0)
        splits.append((c0, g))
        c0 += g
    mesh = plsc.VectorSubcoreMesh(core_axis_name="c", subcore_axis_name="s")

    scratch = [pltpu.VMEM((rows_per_w, n_cols), jnp.int32)]
    scratch += [pltpu.VMEM((n_cols, D_MODEL), jnp.float32) for _ in range(_NBUF)]
    scratch += [pltpu.SemaphoreType.DMA for _ in range(2 * _NBUF)]

    @functools.partial(
        pl.kernel,
        mesh=mesh,
        out_type=jax.ShapeDtypeStruct((n_rows, n_cols, D_MODEL), jnp.float32),
        scratch_types=scratch,
        compiler_params=pltpu.CompilerParams(use_tc_tiling_on_sc=False),
    )
    def lookup(table_hbm, idx_hbm, out_hbm, idx_v, *bufs):
        rows = bufs[:_NBUF]
        sem_in = bufs[_NBUF:2 * _NBUF]
        sem_out = bufs[2 * _NBUF:]
        wid = lax.axis_index("s") * _NC + lax.axis_index("c")
        base = wid * rows_per_w
        pltpu.sync_copy(idx_hbm.at[pl.ds(base, rows_per_w)], idx_v)

        def fire_gather(g, b):
            for (c0, gw) in splits:
                pltpu.async_copy(
                    table_hbm.at[idx_v.at[g, pl.ds(c0, gw)]],
                    rows[b].at[pl.ds(c0, gw)],
                    sem_in[b],
                )

        def wait_gather(b):
            pltpu.make_async_copy(
                table_hbm.at[pl.ds(0, n_cols)], rows[b], sem_in[b]
            ).wait()

        def fire_writeback(g, b):
            pltpu.async_copy(rows[b], out_hbm.at[base + g], sem_out[b])

        def wait_writeback(b):
            pltpu.make_async_copy(rows[b], out_hbm.at[0], sem_out[b]).wait()

        for g in range(_LEAD):
            fire_gather(g, g % _NBUF)

        def superstep(c, _):
            for b in range(_NBUF):
                g = c * _NBUF + b
                gf = g + _LEAD
                bf = (b + _LEAD) % _NBUF

                @pl.when(gf < rows_per_w)
                def _fire():
                    @pl.when(gf >= _NBUF)
                    def _wb():
                        wait_writeback(bf)

                    fire_gather(gf, bf)

                wait_gather(b)
                fire_writeback(g, b)
            return 0

        lax.fori_loop(0, rows_per_w // _NBUF, superstep, 0)

        for b in range(_NBUF):
            wait_writeback(b)

    return lookup


def kernel(x, embedding):
    n_rows, n_cols = x.shape
    vocab = embedding.shape[0]
    idx = jnp.clip(x.astype(jnp.int32), 0, vocab - 1)
    table = embedding * jnp.float32(2.0)
    out = _make_lookup(vocab, n_rows, n_cols)(table, idx)
    return out * jnp.float32(4.0)


# speedup vs baseline: 1.9374x; 1.9374x over previous
import functools
import math

import jax
import jax.numpy as jnp
from jax import lax
from jax.experimental import pallas as pl
from jax.experimental.pallas import tpu as pltpu
from jax.experimental.pallas import tpu_sc as plsc

D_MODEL = 64
SCALE = math.sqrt(D_MODEL)

_NC = 2
_NS = 16
_L = 16
_NW = _NC * _NS

_NBUF = 4
_LEAD = 2
_GMAX = 128


@functools.lru_cache(maxsize=None)
def _make_lookup(vocab, n_rows, n_cols):
    rows_per_w = n_rows // _NW
    splits = []
    c0 = 0
    while c0 < n_cols:
        g = min(_GMAX, n_cols - c0)
        splits.append((c0, g))
        c0 += g
    mesh = plsc.VectorSubcoreMesh(core_axis_name="c", subcore_axis_name="s")

    scratch = [pltpu.VMEM((rows_per_w, n_cols), jnp.int32)]
    scratch += [pltpu.VMEM((n_cols, D_MODEL), jnp.float32) for _ in range(_NBUF)]
    scratch += [pltpu.SemaphoreType.DMA for _ in range(2 * _NBUF)]

    @functools.partial(
        pl.kernel,
        mesh=mesh,
        out_type=jax.ShapeDtypeStruct((n_rows, n_cols, 2 * D_MODEL), jnp.float32),
        scratch_types=scratch,
        compiler_params=pltpu.CompilerParams(use_tc_tiling_on_sc=False),
    )
    def lookup(table_hbm, idx_hbm, out_hbm, idx_v, *bufs):
        rows = bufs[:_NBUF]
        sem_in = bufs[_NBUF:2 * _NBUF]
        sem_out = bufs[2 * _NBUF:]
        wid = lax.axis_index("s") * _NC + lax.axis_index("c")
        base = wid * rows_per_w
        pltpu.sync_copy(idx_hbm.at[pl.ds(base, rows_per_w)], idx_v)

        def fire_gather(g, b):
            for (c0, gw) in splits:
                pltpu.async_copy(
                    table_hbm.at[idx_v.at[g, pl.ds(c0, gw)]],
                    rows[b].at[pl.ds(c0, gw)],
                    sem_in[b],
                )

        def wait_gather(b):
            pltpu.make_async_copy(
                table_hbm.at[pl.ds(0, n_cols)], rows[b], sem_in[b]
            ).wait()

        def fire_writeback(g, b):
            pltpu.async_copy(
                rows[b], out_hbm.at[base + g, :, pl.ds(0, D_MODEL)], sem_out[b]
            )

        def wait_writeback(b):
            pltpu.make_async_copy(
                rows[b], out_hbm.at[0, :, pl.ds(0, D_MODEL)], sem_out[b]
            ).wait()

        for g in range(_LEAD):
            fire_gather(g, g % _NBUF)

        def superstep(c, _):
            for b in range(_NBUF):
                g = c * _NBUF + b
                gf = g + _LEAD
                bf = (b + _LEAD) % _NBUF

                @pl.when(gf < rows_per_w)
                def _fire():
                    @pl.when(gf >= _NBUF)
                    def _wb():
                        wait_writeback(bf)

                    fire_gather(gf, bf)

                wait_gather(b)

                @plsc.parallel_loop(0, n_cols, step=1, unroll=8)
                def _scale(i):
                    for j in range(D_MODEL // _L):
                        sl = pl.ds(j * _L, _L)
                        rows[b][i, sl] = rows[b][i, sl] * SCALE

                fire_writeback(g, b)
            return 0

        lax.fori_loop(0, rows_per_w // _NBUF, superstep, 0)

        for b in range(_NBUF):
            wait_writeback(b)

    return lookup


def kernel(x, embedding):
    n_rows, n_cols = x.shape
    vocab = embedding.shape[0]
    idx = jnp.clip(x.astype(jnp.int32), 0, vocab - 1)
    out = _make_lookup(vocab, n_rows, n_cols)(embedding, idx)
    return out[:, :, :D_MODEL]
